# baseline (speedup 1.0000x reference)
import functools

import jax
import jax.numpy as jnp
from jax import lax
from jax.experimental import pallas as pl
from jax.experimental.pallas import tpu as pltpu
from jax.experimental.pallas import tpu_sc as plsc

B = 4096
S = 200
E = 64
L = 16
NC = 2
NS = 16
NW = NC * NS
TPW = B * S // NW
SW = B // NW
NBUF = 4
DEPTH = 2
SCALE = 8.0


def _sc_gather(src, token_table):
    mesh = plsc.VectorSubcoreMesh(core_axis_name="c", subcore_axis_name="s")

    @functools.partial(
        pl.kernel,
        mesh=mesh,
        compiler_params=pltpu.CompilerParams(use_tc_tiling_on_sc=False),
        out_type=jax.ShapeDtypeStruct((B * S, E), jnp.float32),
        scratch_types=[
            pltpu.VMEM((SW, S), jnp.int32),
            pltpu.VMEM((NBUF, S, E), jnp.float32),
            [pltpu.SemaphoreType.DMA] * NBUF,
            [pltpu.SemaphoreType.DMA] * NBUF,
        ],
    )
    def body(idx_hbm, tab_hbm, out_hbm, idx_v, ring, gsems, osems):
        w = lax.axis_index("s") * NC + lax.axis_index("c")
        q0 = w * SW
        pltpu.sync_copy(idx_hbm.at[pl.ds(q0, SW)], idx_v)

        def start_gather(g, slot):
            pltpu.async_copy(
                tab_hbm.at[idx_v.at[g, pl.ds(0, 128)]],
                ring.at[slot, pl.ds(0, 128)], gsems[slot])
            pltpu.async_copy(
                tab_hbm.at[idx_v.at[g, pl.ds(128, S - 128)]],
                ring.at[slot, pl.ds(128, S - 128)], gsems[slot])

        def wait_gather(slot):
            pltpu.make_async_copy(
                tab_hbm.at[pl.ds(0, S)], ring.at[slot], gsems[slot]).wait()

        t0 = w * TPW

        def out_rows(g):
            return out_hbm.at[pl.ds(t0 + g * S, S)]

        def wait_write(slot):
            pltpu.make_async_copy(
                ring.at[slot], out_rows(0), osems[slot]).wait()

        for k in range(DEPTH):
            start_gather(k, k)

        @pl.loop(0, SW, step=NBUF)
        def _groups(j):
            for k in range(NBUF):
                g = j + k
                wait_gather(k)
                pltpu.async_copy(ring.at[k], out_rows(g), osems[k])

                nxt = g + DEPTH
                nslot = (k + DEPTH) % NBUF

                @pl.when(nxt < SW)
                def _():
                    @pl.when(g >= NBUF - DEPTH)
                    def _():
                        wait_write(nslot)

                    start_gather(nxt, nslot)

        for k in range(NBUF):
            wait_write(k)

    return body(src, token_table)


def _finalize(g2, pos_table):
    BB = 64

    def body(g_ref, p_ref, o_ref):
        g = g_ref[...].reshape(BB, S, E)
        o_ref[...] = g * SCALE + p_ref[...][None]

    return pl.pallas_call(
        body,
        grid=(B // BB,),
        in_specs=[
            pl.BlockSpec((BB * S, E), lambda i: (i, 0)),
            pl.BlockSpec((S, E), lambda i: (0, 0)),
        ],
        out_specs=pl.BlockSpec((BB, S, E), lambda i: (i, 0, 0)),
        out_shape=jax.ShapeDtypeStruct((B, S, E), jnp.float32),
    )(g2, pos_table)


def kernel(src, tgt, token_table, pos_table):
    del tgt
    g2 = _sc_gather(src, token_table)
    return _finalize(g2, pos_table)
